# baseline (speedup 1.0000x reference)
import functools

import jax
import jax.numpy as jnp
from jax import lax
from jax.experimental import pallas as pl
from jax.experimental.pallas import tpu as pltpu
from jax.experimental.pallas import tpu_sc as plsc

BATCH = 16384
D = 64
NUM_CORES = 2
NUM_SUBCORES = 16
NW = NUM_CORES * NUM_SUBCORES
BPW = BATCH // NW
CHUNK = 128
NCH = BPW // CHUNK
NBUF = 2


def _sc_partials(user_idx, pos_idx, neg_idx, user_2w, item_2w):
    mesh = plsc.VectorSubcoreMesh(core_axis_name="c", subcore_axis_name="s")

    @functools.partial(
        pl.kernel,
        mesh=mesh,
        out_type=jax.ShapeDtypeStruct((BATCH * 16,), jnp.float32),
        scratch_types=[
            pltpu.VMEM((BPW,), jnp.int32),
            pltpu.VMEM((BPW,), jnp.int32),
            pltpu.VMEM((BPW,), jnp.int32),
            pltpu.VMEM((BPW,), jnp.int32),
            pltpu.VMEM((BPW,), jnp.int32),
            pltpu.VMEM((BPW,), jnp.int32),
            pltpu.VMEM((NBUF, CHUNK, 2 * D), jnp.float32),
            pltpu.VMEM((NBUF, CHUNK, 2 * D), jnp.float32),
            pltpu.VMEM((NBUF, CHUNK, 2 * D), jnp.float32),
            pltpu.VMEM((BPW * 16,), jnp.float32),
            pltpu.SemaphoreType.DMA,
            pltpu.SemaphoreType.DMA,
        ],
    )
    def k(uidx_h, pidx_h, nidx_h, uemb_h, iemb_h, out_h,
          uidx_v, pidx_v, nidx_v, gu, gp, gn,
          ubuf, pbuf, nbuf, partials, sem0, sem1):
        wid = lax.axis_index("s") * NUM_CORES + lax.axis_index("c")
        base = wid * BPW

        pltpu.sync_copy(uidx_h.at[pl.ds(base, BPW)], uidx_v)
        pltpu.sync_copy(pidx_h.at[pl.ds(base, BPW)], pidx_v)
        pltpu.sync_copy(nidx_h.at[pl.ds(base, BPW)], nidx_v)

        def gidx_body(i, carry):
            sl = pl.ds(i * 16, 16)
            one = jnp.full((16,), 1, jnp.int32)
            gu[sl] = lax.shift_right_logical(uidx_v[sl], one)
            gp[sl] = lax.shift_right_logical(pidx_v[sl], one)
            gn[sl] = lax.shift_right_logical(nidx_v[sl], one)
            return carry

        lax.fori_loop(0, BPW // 16, gidx_body, 0)

        sems = (sem0, sem1)

        def fire(j):
            s = j % NBUF
            sl = pl.ds(j * CHUNK, CHUNK)
            return (
                pltpu.async_copy(uemb_h.at[gu.at[sl]], ubuf.at[s], sems[s]),
                pltpu.async_copy(iemb_h.at[gp.at[sl]], pbuf.at[s], sems[s]),
                pltpu.async_copy(iemb_h.at[gn.at[sl]], nbuf.at[s], sems[s]),
            )

        inflight = {0: fire(0)}
        for j in range(NCH):
            if j + 1 < NCH:
                inflight[j + 1] = fire(j + 1)
            for h in inflight.pop(j):
                h.wait()
            s = j % NBUF

            def group_body(g, carry, j=j, s=s):
                rbase = g * 16
                b0 = j * CHUNK + rbase
                one = jnp.full((16,), 1, jnp.int32)
                hu_vec = (uidx_v[pl.ds(b0, 16)] & one) * D
                hp_vec = (pidx_v[pl.ds(b0, 16)] & one) * D
                hn_vec = (nidx_v[pl.ds(b0, 16)] & one) * D
                for i in range(16):
                    r = rbase + i
                    hu, hp, hn = hu_vec[i], hp_vec[i], hn_vec[i]
                    acc = jnp.zeros((16,), jnp.float32)
                    for k2 in range(D // 16):
                        du = ubuf[s, r, pl.ds(hu + k2 * 16, 16)]
                        dp = pbuf[s, r, pl.ds(hp + k2 * 16, 16)]
                        dn = nbuf[s, r, pl.ds(hn + k2 * 16, 16)]
                        acc = acc + du * (dp - dn)
                    partials[pl.ds((b0 + i) * 16, 16)] = acc
                return carry

            lax.fori_loop(0, CHUNK // 16, group_body, 0)

        pltpu.sync_copy(partials, out_h.at[pl.ds(base * 16, BPW * 16)])

    return k(user_idx, pos_idx, neg_idx, user_2w, item_2w)


def _tc_loss(partials_2d):
    def body(x_ref, o_ref):
        r = lax.broadcasted_iota(jnp.int32, (128, 128), 0) // 16
        c = lax.broadcasted_iota(jnp.int32, (128, 128), 1) // 16
        m = (r == c).astype(jnp.float32)
        y = jnp.dot(x_ref[...], m, preferred_element_type=jnp.float32,
                    precision=jax.lax.Precision.HIGHEST)
        t = -y
        sp = jnp.maximum(t, 0.0) + jnp.log(1.0 + jnp.exp(-jnp.abs(t)))
        o_ref[0, 0] = jnp.sum(sp) / (16.0 * BATCH)

    out = pl.pallas_call(
        body,
        out_shape=jax.ShapeDtypeStruct((1, 1), jnp.float32),
        out_specs=pl.BlockSpec(memory_space=pltpu.SMEM),
    )(partials_2d)
    return out[0, 0]


def kernel(user_idx, pos_idx, neg_idx, user_emb, item_emb):
    user_2w = user_emb.reshape(-1, 2 * D)
    item_2w = item_emb.reshape(-1, 2 * D)
    partials = _sc_partials(user_idx, pos_idx, neg_idx, user_2w, item_2w)
    return _tc_loss(partials.reshape(2048, 128))


# speedup vs baseline: 1.2742x; 1.2742x over previous
import functools

import jax
import jax.numpy as jnp
from jax import lax
from jax.experimental import pallas as pl
from jax.experimental.pallas import tpu as pltpu
from jax.experimental.pallas import tpu_sc as plsc

BATCH = 16384
D = 64
VOCAB = 100000
NUM_CORES = 2
NUM_SUBCORES = 16
NW = NUM_CORES * NUM_SUBCORES
DPW = D // NW
HALF = BATCH // 2


def _sc_transposed_gather(user_idx, pos_idx, neg_idx, user_t, item_t):
    mesh = plsc.VectorSubcoreMesh(core_axis_name="c", subcore_axis_name="s")

    @functools.partial(
        pl.kernel,
        mesh=mesh,
        out_type=(
            jax.ShapeDtypeStruct((D, BATCH), jnp.float32),
            jax.ShapeDtypeStruct((D, BATCH), jnp.float32),
        ),
        compiler_params=pltpu.CompilerParams(needs_layout_passes=False),
        scratch_types=[
            pltpu.VMEM((VOCAB,), jnp.float32),
            pltpu.VMEM((HALF,), jnp.int32),
            pltpu.VMEM((HALF,), jnp.float32),
            pltpu.VMEM((HALF,), jnp.float32),
            pltpu.SemaphoreType.DMA,
        ],
    )
    def k(uidx_h, pidx_h, nidx_h, ut_h, it_h, out_u, out_d,
          line, idxb, oa, ob, sem):
        wid = lax.axis_index("s") * NUM_CORES + lax.axis_index("c")

        def gather_into(dst, carry_unused):
            def body(i, carry):
                sl = pl.ds(i * 16, 16)
                dst[sl] = plsc.load_gather(line, [idxb[sl]])
                return carry
            lax.fori_loop(0, HALF // 16, body, 0, unroll=8)

        def gather_sub(dst, carry_unused):
            def body(i, carry):
                sl = pl.ds(i * 16, 16)
                dst[sl] = dst[sl] - plsc.load_gather(line, [idxb[sl]])
                return carry
            lax.fori_loop(0, HALF // 16, body, 0, unroll=8)

        for t in range(DPW):
            c = wid * DPW + t
            pltpu.async_copy(ut_h.at[c], line, sem).wait()
            for h in range(2):
                hsl = pl.ds(h * HALF, HALF)
                pltpu.sync_copy(uidx_h.at[hsl], idxb)
                gather_into(oa, 0)
                pltpu.sync_copy(oa, out_u.at[c, hsl])

        for t in range(DPW):
            c = wid * DPW + t
            pltpu.async_copy(it_h.at[c], line, sem).wait()
            for h in range(2):
                hsl = pl.ds(h * HALF, HALF)
                pltpu.sync_copy(pidx_h.at[hsl], idxb)
                gather_into(oa, 0)
                pltpu.sync_copy(nidx_h.at[hsl], idxb)
                gather_sub(oa, 0)
                pltpu.sync_copy(oa, out_d.at[c, hsl])

    return k(user_idx, pos_idx, neg_idx, user_t, item_t)


def _tc_loss(u_t, d_t):
    def body(u_ref, d_ref, o_ref):
        x = jnp.sum(u_ref[...] * d_ref[...], axis=0, keepdims=True)
        t = -x
        sp = jnp.maximum(t, 0.0) + jnp.log(1.0 + jnp.exp(-jnp.abs(t)))
        o_ref[0, 0] = jnp.sum(sp) / BATCH

    out = pl.pallas_call(
        body,
        out_shape=jax.ShapeDtypeStruct((1, 1), jnp.float32),
        out_specs=pl.BlockSpec(memory_space=pltpu.SMEM),
    )(u_t, d_t)
    return out[0, 0]


def kernel(user_idx, pos_idx, neg_idx, user_emb, item_emb):
    user_t = user_emb.T
    item_t = item_emb.T
    u_t, d_t = _sc_transposed_gather(user_idx, pos_idx, neg_idx, user_t, item_t)
    return _tc_loss(u_t, d_t)


# speedup vs baseline: 2.0623x; 1.6185x over previous
import functools

import jax
import jax.numpy as jnp
from jax import lax
from jax.experimental import pallas as pl
from jax.experimental.pallas import tpu as pltpu
from jax.experimental.pallas import tpu_sc as plsc

BATCH = 16384
D = 64
VOCAB = 100000
NUM_CORES = 2
NUM_SUBCORES = 16
NW = NUM_CORES * NUM_SUBCORES
DPW = D // NW
HALF = BATCH // 2


def _sc_transposed_gather(user_idx, pos_idx, neg_idx, user_t, item_t):
    mesh = plsc.VectorSubcoreMesh(core_axis_name="c", subcore_axis_name="s")

    @functools.partial(
        pl.kernel,
        mesh=mesh,
        out_type=(
            jax.ShapeDtypeStruct((D, BATCH), jnp.float32),
            jax.ShapeDtypeStruct((D, BATCH), jnp.float32),
        ),
        compiler_params=pltpu.CompilerParams(needs_layout_passes=False),
        scratch_types=[
            pltpu.VMEM((VOCAB,), jnp.float32),
            pltpu.VMEM((HALF,), jnp.int32),
            pltpu.VMEM((HALF,), jnp.int32),
            pltpu.VMEM((HALF,), jnp.float32),
            pltpu.SemaphoreType.DMA,
        ],
    )
    def k(uidx_h, pidx_h, nidx_h, ut_h, it_h, out_u, out_d,
          line, idxp, idxn, oa, sem):
        wid = lax.axis_index("s") * NUM_CORES + lax.axis_index("c")

        for t in range(DPW):
            c = wid * DPW + t
            pltpu.async_copy(ut_h.at[c], line, sem).wait()
            for h in range(2):
                hsl = pl.ds(h * HALF, HALF)
                pltpu.sync_copy(uidx_h.at[hsl], idxp)

                @plsc.parallel_loop(0, HALF // 16, unroll=16)
                def ubody(i):
                    sl = pl.ds(i * 16, 16)
                    oa[sl] = plsc.load_gather(line, [idxp[sl]])

                pltpu.sync_copy(oa, out_u.at[c, hsl])

        for t in range(DPW):
            c = wid * DPW + t
            pltpu.async_copy(it_h.at[c], line, sem).wait()
            for h in range(2):
                hsl = pl.ds(h * HALF, HALF)
                pltpu.sync_copy(pidx_h.at[hsl], idxp)
                pltpu.sync_copy(nidx_h.at[hsl], idxn)

                @plsc.parallel_loop(0, HALF // 16, unroll=16)
                def ibody(i):
                    sl = pl.ds(i * 16, 16)
                    gp = plsc.load_gather(line, [idxp[sl]])
                    gn = plsc.load_gather(line, [idxn[sl]])
                    oa[sl] = gp - gn

                pltpu.sync_copy(oa, out_d.at[c, hsl])

    return k(user_idx, pos_idx, neg_idx, user_t, item_t)


def _tc_loss(u_t, d_t):
    def body(u_ref, d_ref, o_ref):
        x = jnp.sum(u_ref[...] * d_ref[...], axis=0, keepdims=True)
        t = -x
        sp = jnp.maximum(t, 0.0) + jnp.log(1.0 + jnp.exp(-jnp.abs(t)))
        o_ref[0, 0] = jnp.sum(sp) / BATCH

    out = pl.pallas_call(
        body,
        out_shape=jax.ShapeDtypeStruct((1, 1), jnp.float32),
        out_specs=pl.BlockSpec(memory_space=pltpu.SMEM),
    )(u_t, d_t)
    return out[0, 0]


def kernel(user_idx, pos_idx, neg_idx, user_emb, item_emb):
    user_t = user_emb.T
    item_t = item_emb.T
    u_t, d_t = _sc_transposed_gather(user_idx, pos_idx, neg_idx, user_t, item_t)
    return _tc_loss(u_t, d_t)
